# initial kernel scaffold (unmeasured)
import jax
import jax.numpy as jnp
from jax import lax
from jax.experimental import pallas as pl
from jax.experimental.pallas import tpu as pltpu


def kernel(
    x,
):
    def body(*refs):
        pass

    out_shape = jax.ShapeDtypeStruct(..., jnp.float32)
    return pl.pallas_call(body, out_shape=out_shape)(...)


# baseline (device time: 1176344 ns/iter reference)
import jax
import jax.numpy as jnp
from jax import lax
from jax.experimental import pallas as pl
from jax.experimental.pallas import tpu as pltpu

N_Y = 4


def kernel(x):
    m_per, n = x.shape
    half = m_per // 2

    def body(x_ref, out_ref, copy_sem, send_sems, recv_sems):
        my_x = lax.axis_index("x")
        my_y = lax.axis_index("y")
        my_z = lax.axis_index("z")
        right = (my_y + 1) % N_Y
        left = (my_y + N_Y - 1) % N_Y

        barrier = pltpu.get_barrier_semaphore()
        for nbr in (left, right):
            pl.semaphore_signal(
                barrier,
                inc=1,
                device_id=(my_x, nbr, my_z),
                device_id_type=pl.DeviceIdType.MESH,
            )
        pl.semaphore_wait(barrier, 2)

        local = pltpu.make_async_copy(
            x_ref, out_ref.at[pl.ds(my_y * m_per, m_per), :], copy_sem
        )
        local.start()

        for h in range(N_Y - 1):
            fwd_origin = (my_y - h) % N_Y
            bwd_origin = (my_y + h) % N_Y
            if h == 0:
                fwd_src = x_ref.at[pl.ds(0, half), :]
                bwd_src = x_ref.at[pl.ds(half, half), :]
            else:
                fwd_src = out_ref.at[pl.ds(fwd_origin * m_per, half), :]
                bwd_src = out_ref.at[pl.ds(bwd_origin * m_per + half, half), :]

            fwd = pltpu.make_async_remote_copy(
                src_ref=fwd_src,
                dst_ref=out_ref.at[pl.ds(fwd_origin * m_per, half), :],
                send_sem=send_sems.at[0, h],
                recv_sem=recv_sems.at[0, h],
                device_id=(my_x, right, my_z),
                device_id_type=pl.DeviceIdType.MESH,
            )
            bwd = pltpu.make_async_remote_copy(
                src_ref=bwd_src,
                dst_ref=out_ref.at[pl.ds(bwd_origin * m_per + half, half), :],
                send_sem=send_sems.at[1, h],
                recv_sem=recv_sems.at[1, h],
                device_id=(my_x, left, my_z),
                device_id_type=pl.DeviceIdType.MESH,
            )
            fwd.start()
            bwd.start()
            fwd.wait()
            bwd.wait()

        local.wait()

    return pl.pallas_call(
        body,
        out_shape=jax.ShapeDtypeStruct((N_Y * m_per, n), x.dtype),
        in_specs=[pl.BlockSpec(memory_space=pl.ANY)],
        out_specs=pl.BlockSpec(memory_space=pl.ANY),
        scratch_shapes=[
            pltpu.SemaphoreType.DMA,
            pltpu.SemaphoreType.DMA((2, N_Y - 1)),
            pltpu.SemaphoreType.DMA((2, N_Y - 1)),
        ],
        compiler_params=pltpu.CompilerParams(collective_id=0),
    )(x)


# device time: 1110338 ns/iter; 1.0594x vs baseline; 1.0594x over previous
import jax
import jax.numpy as jnp
from jax import lax
from jax.experimental import pallas as pl
from jax.experimental.pallas import tpu as pltpu

N_Y = 4
S = 1
_MESH = pl.DeviceIdType.MESH


def kernel(x):
    m_per, n = x.shape
    half = m_per // 2
    piece = half // S

    def body(x_ref, out_ref, copy_sem, fs, fr, bs, br, xs, xr):
        my_x = lax.axis_index("x")
        my_y = lax.axis_index("y")
        my_z = lax.axis_index("z")
        partner = 1 - my_x
        right = (my_y + 1) % N_Y
        left = (my_y + N_Y - 1) % N_Y
        hoff = my_x * half

        has_left = my_y > 0
        has_right = my_y < N_Y - 1

        barrier = pltpu.get_barrier_semaphore()
        pl.semaphore_signal(
            barrier, inc=1, device_id=(partner, my_y, my_z),
            device_id_type=_MESH,
        )

        @pl.when(has_left)
        def _():
            pl.semaphore_signal(
                barrier, inc=1, device_id=(my_x, left, my_z),
                device_id_type=_MESH,
            )

        @pl.when(has_right)
        def _():
            pl.semaphore_signal(
                barrier, inc=1, device_id=(my_x, right, my_z),
                device_id_type=_MESH,
            )

        @pl.when(has_left & has_right)
        def _():
            pl.semaphore_wait(barrier, 3)

        @pl.when(jnp.logical_not(has_left & has_right))
        def _():
            pl.semaphore_wait(barrier, 2)

        local = pltpu.make_async_copy(
            x_ref, out_ref.at[pl.ds(my_y * m_per, m_per), :], copy_sem
        )
        local.start()

        def csl(origin, p):
            return pl.ds(origin * m_per + hoff + p * piece, piece)

        desc_f, desc_b = {}, {}
        for s in range(N_Y - 1):
            for p in range(S):
                of = (my_y - s) % N_Y
                ob = (my_y + s) % N_Y
                own_src = x_ref.at[pl.ds(hoff + p * piece, piece), :]
                desc_f[s, p] = pltpu.make_async_remote_copy(
                    src_ref=own_src if s == 0 else out_ref.at[csl(of, p), :],
                    dst_ref=out_ref.at[csl(of, p), :],
                    send_sem=fs.at[s, p], recv_sem=fr.at[s, p],
                    device_id=(my_x, right, my_z), device_id_type=_MESH,
                )
                desc_b[s, p] = pltpu.make_async_remote_copy(
                    src_ref=own_src if s == 0 else out_ref.at[csl(ob, p), :],
                    dst_ref=out_ref.at[csl(ob, p), :],
                    send_sem=bs.at[s, p], recv_sem=br.at[s, p],
                    device_id=(my_x, left, my_z), device_id_type=_MESH,
                )
        desc_x = {}
        for t in range(N_Y - 1):
            for p in range(S):
                of = (my_y - 1 - t) % N_Y
                ob = (my_y + 1 + t) % N_Y
                desc_x[2 * t, p] = pltpu.make_async_remote_copy(
                    src_ref=out_ref.at[csl(of, p), :],
                    dst_ref=out_ref.at[csl(of, p), :],
                    send_sem=xs.at[2 * t, p], recv_sem=xr.at[2 * t, p],
                    device_id=(partner, my_y, my_z), device_id_type=_MESH,
                )
                desc_x[2 * t + 1, p] = pltpu.make_async_remote_copy(
                    src_ref=out_ref.at[csl(ob, p), :],
                    dst_ref=out_ref.at[csl(ob, p), :],
                    send_sem=xs.at[2 * t + 1, p], recv_sem=xr.at[2 * t + 1, p],
                    device_id=(partner, my_y, my_z), device_id_type=_MESH,
                )

        for p in range(S):
            @pl.when(has_right)
            def _(p=p):
                desc_f[0, p].start()

            @pl.when(has_left)
            def _(p=p):
                desc_b[0, p].start()

        for t in range(N_Y - 1):
            recv_f = my_y >= t + 1
            recv_b = my_y <= N_Y - 2 - t
            for p in range(S):
                @pl.when(recv_f)
                def _(t=t, p=p):
                    desc_f[t, p].wait_recv()

                if t + 1 <= N_Y - 2:
                    @pl.when(recv_f & has_right)
                    def _(t=t, p=p):
                        desc_f[t + 1, p].start()

                @pl.when(recv_f)
                def _(t=t, p=p):
                    desc_x[2 * t, p].start()

                @pl.when(recv_b)
                def _(t=t, p=p):
                    desc_b[t, p].wait_recv()

                if t + 1 <= N_Y - 2:
                    @pl.when(recv_b & has_left)
                    def _(t=t, p=p):
                        desc_b[t + 1, p].start()

                @pl.when(recv_b)
                def _(t=t, p=p):
                    desc_x[2 * t + 1, p].start()

        for t in range(N_Y - 1):
            recv_f = my_y >= t + 1
            recv_b = my_y <= N_Y - 2 - t
            for p in range(S):
                @pl.when(recv_f)
                def _(t=t, p=p):
                    desc_x[2 * t, p].wait()

                @pl.when(recv_b)
                def _(t=t, p=p):
                    desc_x[2 * t + 1, p].wait()

        for s in range(N_Y - 1):
            send_f = has_right & (my_y >= s)
            send_b = has_left & (my_y <= N_Y - 1 - s)
            for p in range(S):
                @pl.when(send_f)
                def _(s=s, p=p):
                    desc_f[s, p].wait_send()

                @pl.when(send_b)
                def _(s=s, p=p):
                    desc_b[s, p].wait_send()

        local.wait()

    return pl.pallas_call(
        body,
        out_shape=jax.ShapeDtypeStruct((N_Y * m_per, n), x.dtype),
        in_specs=[pl.BlockSpec(memory_space=pl.ANY)],
        out_specs=pl.BlockSpec(memory_space=pl.ANY),
        scratch_shapes=[
            pltpu.SemaphoreType.DMA,
            pltpu.SemaphoreType.DMA((N_Y - 1, S)),
            pltpu.SemaphoreType.DMA((N_Y - 1, S)),
            pltpu.SemaphoreType.DMA((N_Y - 1, S)),
            pltpu.SemaphoreType.DMA((N_Y - 1, S)),
            pltpu.SemaphoreType.DMA((2 * (N_Y - 1), S)),
            pltpu.SemaphoreType.DMA((2 * (N_Y - 1), S)),
        ],
        compiler_params=pltpu.CompilerParams(collective_id=0),
    )(x)
